# Initial kernel scaffold; baseline (speedup 1.0000x reference)
import jax
import jax.numpy as jnp
from jax.experimental import pallas as pl


def kernel(X, feature_set, ctx_indices):
    raise NotImplementedError("write your pallas kernel here")


# speedup vs baseline: 8.4764x; 8.4764x over previous
import jax
import jax.numpy as jnp
from jax.experimental import pallas as pl


def _expand_sub_kernel(x_ref, fs_ref, m_ref, o_ref):
    x = x_ref[0]
    q, d = x.shape
    n_w = fs_ref.shape[0]
    xb = jnp.broadcast_to(x[:, None, :], (q, n_w, d))
    fsb = jnp.broadcast_to(fs_ref[...][None, :, :], (q, n_w, d))
    m = jnp.broadcast_to(m_ref[...][None, :, :] != 0, (q, n_w, d))
    o_ref[0] = jnp.where(m, fsb, xb)


def kernel(X, feature_set, ctx_indices):
    batch = X.shape[:-2]
    q, d = X.shape[-2], X.shape[-1]
    n_w, d_ctx = feature_set.shape
    Xf = X.reshape((-1, q, d))
    nb = Xf.shape[0]

    fsrow = jnp.zeros((n_w, d), dtype=X.dtype).at[:, ctx_indices].set(feature_set)
    mask = jnp.zeros((1, d), dtype=jnp.int32).at[0, ctx_indices].set(1)

    out = pl.pallas_call(
        _expand_sub_kernel,
        grid=(nb,),
        in_specs=[
            pl.BlockSpec((1, q, d), lambda b: (b, 0, 0)),
            pl.BlockSpec((n_w, d), lambda b: (0, 0)),
            pl.BlockSpec((1, d), lambda b: (0, 0)),
        ],
        out_specs=pl.BlockSpec((1, q, n_w, d), lambda b: (b, 0, 0, 0)),
        out_shape=jax.ShapeDtypeStruct((nb, q, n_w, d), X.dtype),
    )(Xf, fsrow, mask)
    return out.reshape(batch + (q * n_w, d))
